# baseline (speedup 1.0000x reference)
import jax
import jax.numpy as jnp
from jax import lax
from jax.experimental import pallas as pl
from jax.experimental.pallas import tpu as pltpu
from jax.experimental.pallas import tpu_sc as plsc

_NUM_CORES = 1
_NUM_SUBCORES = 8
_NW = _NUM_CORES * _NUM_SUBCORES
_L = 16


def _sc_gather_body(idx_hbm, w_hbm, out_hbm, idx_v, out_v, w_v):
    rows, fi = idx_v.shape
    wid = lax.axis_index("s") * _NUM_CORES + lax.axis_index("c")
    base = wid * rows
    pltpu.sync_copy(w_hbm, w_v)
    pltpu.sync_copy(idx_hbm.at[pl.ds(base, rows), pl.ds(0, fi)], idx_v)

    n_vecs = fi // _L

    @plsc.parallel_loop(0, rows * n_vecs, unroll=8)
    def _(c):
        r = c // n_vecs
        col = (c % n_vecs) * _L
        idx = idx_v[r, pl.ds(col, _L)]
        out_v[r, pl.ds(col, _L)] = plsc.load_gather(w_v, [idx])

    pltpu.sync_copy(out_v, out_hbm.at[pl.ds(base, rows)])


def _tc_matmul_body(a_ref, weff_ref, hb_ref, w_ref, out_ref):
    acc = lax.dot_general(
        a_ref[...], weff_ref[...],
        dimension_numbers=(((1,), (1,)), ((), ())),
        preferred_element_type=jnp.float32,
    )
    wb = jnp.broadcast_to(w_ref[0, :], (hb_ref.shape[0], w_ref.shape[1]))
    bias = jnp.take_along_axis(wb, hb_ref[...], axis=1)[:, 0]
    out_ref[...] = acc + bias[None, :]


def kernel(a, hash_idx, W):
    B, FI = a.shape
    FO = hash_idx.shape[0]
    K = W.shape[0]
    rows_per = FO // _NW

    hash_bias = hash_idx[:, FI:]

    sc_gather = pl.kernel(
        _sc_gather_body,
        out_type=jax.ShapeDtypeStruct((FO, FI), jnp.float32),
        mesh=plsc.VectorSubcoreMesh(
            core_axis_name="c", subcore_axis_name="s", num_cores=1, num_subcores=8),
        compiler_params=pltpu.CompilerParams(needs_layout_passes=False),
        scratch_types=[
            pltpu.VMEM((rows_per, FI), jnp.int32),
            pltpu.VMEM((rows_per, FI), jnp.float32),
            pltpu.VMEM((K,), jnp.float32),
        ],
    )
    weff = sc_gather(hash_idx, W)

    return pl.pallas_call(
        _tc_matmul_body,
        out_shape=jax.ShapeDtypeStruct((B, FO), jnp.float32),
    )(a, weff, hash_bias, W.reshape(1, K))


# speedup vs baseline: 1.0911x; 1.0911x over previous
import jax
import jax.numpy as jnp
from jax import lax
from jax.experimental import pallas as pl
from jax.experimental.pallas import tpu as pltpu
from jax.experimental.pallas import tpu_sc as plsc

_NUM_CORES = 1
_NUM_SUBCORES = 16
_NW = _NUM_CORES * _NUM_SUBCORES
_L = 16


def _sc_gather_body(idx_hbm, w_hbm, out_hbm, idx_v, out_v, w_v):
    rows, fi = idx_v.shape
    wid = lax.axis_index("s") * _NUM_CORES + lax.axis_index("c")
    base = wid * rows
    pltpu.sync_copy(w_hbm, w_v)
    pltpu.sync_copy(idx_hbm.at[pl.ds(base, rows), pl.ds(0, fi)], idx_v)

    n_vecs = fi // _L

    @plsc.parallel_loop(0, rows * n_vecs, unroll=8)
    def _(c):
        r = c // n_vecs
        col = (c % n_vecs) * _L
        idx = idx_v[r, pl.ds(col, _L)]
        out_v[r, pl.ds(col, _L)] = plsc.load_gather(w_v, [idx])

    pltpu.sync_copy(out_v, out_hbm.at[pl.ds(base, rows)])


def _tc_matmul_body(a_ref, weff_ref, hb_ref, w_ref, out_ref):
    acc = lax.dot_general(
        a_ref[...], weff_ref[...],
        dimension_numbers=(((1,), (1,)), ((), ())),
        preferred_element_type=jnp.float32,
    )
    wb = jnp.broadcast_to(w_ref[0, :], (hb_ref.shape[0], w_ref.shape[1]))
    bias = jnp.take_along_axis(wb, hb_ref[...], axis=1)[:, 0]
    out_ref[...] = acc + bias[None, :]


def kernel(a, hash_idx, W):
    B, FI = a.shape
    FO = hash_idx.shape[0]
    K = W.shape[0]
    rows_per = FO // _NW

    hash_bias = hash_idx[:, FI:]

    sc_gather = pl.kernel(
        _sc_gather_body,
        out_type=jax.ShapeDtypeStruct((FO, FI), jnp.float32),
        mesh=plsc.VectorSubcoreMesh(
            core_axis_name="c", subcore_axis_name="s", num_cores=1),
        compiler_params=pltpu.CompilerParams(needs_layout_passes=False),
        scratch_types=[
            pltpu.VMEM((rows_per, FI), jnp.int32),
            pltpu.VMEM((rows_per, FI), jnp.float32),
            pltpu.VMEM((K,), jnp.float32),
        ],
    )
    weff = sc_gather(hash_idx, W)

    return pl.pallas_call(
        _tc_matmul_body,
        out_shape=jax.ShapeDtypeStruct((B, FO), jnp.float32),
    )(a, weff, hash_bias, W.reshape(1, K))
